# baseline (speedup 1.0000x reference)
import functools

import jax
import jax.numpy as jnp
from jax import lax
from jax.experimental import pallas as pl
from jax.experimental import layout as jlayout
from jax.experimental.pallas import tpu as pltpu
from jax.experimental.pallas import tpu_sc as plsc


def _gather_kernel(B, D, CH, K):
    info = plsc.get_sparse_core_info()
    NC, NS = info.num_cores, info.num_subcores
    NW = NC * NS
    b_per_w = B // NW
    n_chunks = b_per_w // CH
    n_pairs = n_chunks // (2 * K)
    assert B % NW == 0 and b_per_w % CH == 0 and n_chunks % (2 * K) == 0
    mesh = plsc.VectorSubcoreMesh(core_axis_name="c", subcore_axis_name="s")

    @functools.partial(
        pl.kernel,
        out_type=jax.ShapeDtypeStruct((B, 2 * D), jnp.float32),
        mesh=mesh,
        scratch_types=[
            pltpu.VMEM((b_per_w,), jnp.int32),
            pltpu.VMEM((2 * K, CH, 2 * D), jnp.float32),
            pltpu.SemaphoreType.DMA,
            pltpu.SemaphoreType.DMA,
            pltpu.SemaphoreType.DMA,
            pltpu.SemaphoreType.DMA,
        ],
        compiler_params=pltpu.CompilerParams(use_tc_tiling_on_sc=False),
    )
    def k(idx_hbm, table_hbm, out_hbm, idx_all, rows, gsem_a, gsem_b, ssem_a, ssem_b):
        wid = lax.axis_index("s") * NC + lax.axis_index("c")
        base = wid * b_per_w
        pltpu.sync_copy(idx_hbm.at[pl.ds(base, b_per_w)], idx_all)

        def gather_desc(g, half, b, sem):
            ch = g * K + b
            idx_sl = idx_all.at[pl.ds(ch * CH, CH)]
            return pltpu.make_async_copy(
                table_hbm.at[idx_sl], rows.at[half * K + b], sem)

        def store_desc(g, half, b, sem):
            ch = g * K + b
            return pltpu.make_async_copy(
                rows.at[half * K + b, pl.ds(0, CH), pl.ds(0, D)],
                out_hbm.at[pl.ds(base + ch * CH, CH), pl.ds(0, D)], sem)

        def fire_gathers(g, half, sem):
            for b in range(K):
                gather_desc(g, half, b, sem).start()

        def drain_gathers(g, half, sem):
            for b in range(K):
                gather_desc(g, half, b, sem).wait()

        def fire_stores(g, half, sem):
            for b in range(K):
                store_desc(g, half, b, sem).start()

        def drain_stores(g, half, sem):
            for b in range(K):
                store_desc(g, half, b, sem).wait()

        @pl.loop(0, n_pairs)
        def _(h):
            g0 = 2 * h
            g1 = 2 * h + 1

            @pl.when(h > 0)
            def _():
                drain_stores(g0 - 2, 0, ssem_a)

            fire_gathers(g0, 0, gsem_a)
            drain_gathers(g0, 0, gsem_a)

            @pl.when(h > 0)
            def _():
                drain_stores(g1 - 2, 1, ssem_b)

            fire_gathers(g1, 1, gsem_b)
            fire_stores(g0, 0, ssem_a)
            drain_gathers(g1, 1, gsem_b)
            fire_stores(g1, 1, ssem_b)

        drain_stores(2 * n_pairs - 2, 0, ssem_a)
        drain_stores(2 * n_pairs - 1, 1, ssem_b)

    return k


@jax.jit
def kernel(data, table):
    B = data.size
    V, D = table.shape
    flat = data.reshape(B)
    tpad = jnp.pad(table, ((0, 0), (0, D)))
    out2 = _gather_kernel(B, D, 128, 2)(flat, tpad)
    return out2[:, :D].reshape(*data.shape, D)


# speedup vs baseline: 1.3879x; 1.3879x over previous
import functools

import jax
import jax.numpy as jnp
from jax import lax
from jax.experimental import pallas as pl
from jax.experimental.pallas import tpu as pltpu
from jax.experimental.pallas import tpu_sc as plsc


def _lookup_kernel(NI, NJ, D, CH):
    info = plsc.get_sparse_core_info()
    NC, NS = info.num_cores, info.num_subcores
    NW = NC * NS
    NB_I = NI // CH
    per_w = NB_I // NW
    n_sub = per_w * NJ
    n_pairs = n_sub // 2
    W = CH * NJ
    PITCH = CH + 1
    assert NI % CH == 0 and NB_I % NW == 0 and NJ % 2 == 0
    mesh = plsc.VectorSubcoreMesh(core_axis_name="c", subcore_axis_name="s")

    @functools.partial(
        pl.kernel,
        out_type=jax.ShapeDtypeStruct((NJ, D // 8, NB_I, 8, CH), jnp.float32),
        mesh=mesh,
        scratch_types=[
            pltpu.VMEM((W,), jnp.int32),
            pltpu.VMEM((CH,), jnp.int32),
            pltpu.VMEM((CH,), jnp.int32),
            pltpu.VMEM((2, CH, D), jnp.float32),
            pltpu.VMEM((2, D, PITCH), jnp.float32),
            pltpu.SemaphoreType.DMA,
            pltpu.SemaphoreType.DMA,
            pltpu.SemaphoreType.DMA,
            pltpu.SemaphoreType.DMA,
        ],
        compiler_params=pltpu.CompilerParams(
            use_tc_tiling_on_sc=False, needs_layout_passes=False),
    )
    def k(idx1d, t64, out, win, p0, p1, grows, oblk, g0, g1, s0, s1):
        wid = lax.axis_index("s") * NC + lax.axis_index("c")
        prefs = (p0, p1)
        gsems = (g0, g1)
        ssems = (s0, s1)
        lanes = lax.iota(jnp.int32, 16)
        lanesj = lanes * NJ

        def decode(n):
            m = n // NJ
            j = n - m * NJ
            return j, wid + m * NW

        def prep(n, s):
            j, ib = decode(n)

            @pl.when(j == 0)
            def _():
                pltpu.sync_copy(idx1d.at[pl.ds(ib * W, W)], win)

            pref = prefs[s]
            for g in range(CH // 16):
                v = plsc.load_gather(win, [lanesj + (g * 16 * NJ + j)])
                pref[pl.ds(g * 16, 16)] = v
            pltpu.async_copy(t64.at[pref], grows.at[s], gsems[s])

        def gather_wait(s):
            pltpu.make_async_copy(
                t64.at[prefs[s]], grows.at[s], gsems[s]).wait()

        def transpose(s):
            gref = grows.at[s]
            oref = oblk.at[s]

            @pl.loop(0, CH)
            def _(kk):
                ksplat = jnp.full((16,), kk, jnp.int32)
                for c in range(D // 16):
                    v = gref[kk, pl.ds(c * 16, 16)]
                    plsc.store_scatter(oref, [lanes + c * 16, ksplat], v)

        def store_start(n, s):
            j, ib = decode(n)
            for tr in range(D // 8):
                pltpu.make_async_copy(
                    oblk.at[s, pl.ds(8 * tr, 8), pl.ds(0, CH)],
                    out.at[j, tr, ib], ssems[s]).start()

        def store_wait(s):
            for tr in range(D // 8):
                pltpu.make_async_copy(
                    oblk.at[s, pl.ds(8 * tr, 8), pl.ds(0, CH)],
                    out.at[0, tr, 0], ssems[s]).wait()

        prep(0, 0)

        @pl.loop(0, n_pairs)
        def _(h):
            n0 = 2 * h
            prep(n0 + 1, 1)
            gather_wait(0)

            @pl.when(h > 0)
            def _():
                store_wait(0)

            transpose(0)
            store_start(n0, 0)

            @pl.when(h < n_pairs - 1)
            def _():
                prep(n0 + 2, 0)

            gather_wait(1)

            @pl.when(h > 0)
            def _():
                store_wait(1)

            transpose(1)
            store_start(n0 + 1, 1)

        store_wait(0)
        store_wait(1)

    return k


@jax.jit
def kernel(data, table):
    NI, NJ = data.shape
    V, D = table.shape
    idx1d = data.reshape(NI * NJ)
    CH = 128
    out_t = _lookup_kernel(NI, NJ, D, CH)(idx1d, table)
    out_phys = out_t.transpose(0, 1, 3, 2, 4).reshape(NJ, D, NI)
    return out_phys.transpose(2, 0, 1)


# speedup vs baseline: 1.4165x; 1.0206x over previous
import functools

import jax
import jax.numpy as jnp
from jax import lax
from jax.experimental import pallas as pl
from jax.experimental.pallas import tpu as pltpu
from jax.experimental.pallas import tpu_sc as plsc


def _lookup_kernel(NI, NJ, D, CH):
    info = plsc.get_sparse_core_info()
    NC, NS = info.num_cores, info.num_subcores
    NW = NC * NS
    NB_I = NI // CH
    per_w = NB_I // NW
    n_sub = per_w * NJ
    n_pairs = n_sub // 2
    W = CH * NJ
    PITCH = CH + 1
    assert NI % CH == 0 and NB_I % NW == 0 and NJ % 2 == 0
    mesh = plsc.VectorSubcoreMesh(core_axis_name="c", subcore_axis_name="s")

    @functools.partial(
        pl.kernel,
        out_type=jax.ShapeDtypeStruct((NJ, D // 8, NB_I, 8, CH), jnp.float32),
        mesh=mesh,
        scratch_types=[
            pltpu.VMEM((W,), jnp.int32),
            pltpu.VMEM((CH,), jnp.int32),
            pltpu.VMEM((CH,), jnp.int32),
            pltpu.VMEM((2, CH, D), jnp.float32),
            pltpu.VMEM((2, D, PITCH), jnp.float32),
            pltpu.SemaphoreType.DMA,
            pltpu.SemaphoreType.DMA,
            pltpu.SemaphoreType.DMA,
            pltpu.SemaphoreType.DMA,
        ],
        compiler_params=pltpu.CompilerParams(
            use_tc_tiling_on_sc=False, needs_layout_passes=False),
    )
    def k(idx1d, t64, out, win, p0, p1, grows, oblk, g0, g1, s0, s1):
        wid = lax.axis_index("s") * NC + lax.axis_index("c")
        prefs = (p0, p1)
        gsems = (g0, g1)
        ssems = (s0, s1)
        lanes = lax.iota(jnp.int32, 16)
        lanesj = lanes * NJ

        def decode(n):
            m = n // NJ
            j = n - m * NJ
            return j, wid + m * NW

        def prep(n, s):
            j, ib = decode(n)

            @pl.when(j == 0)
            def _():
                pltpu.sync_copy(idx1d.at[pl.ds(ib * W, W)], win)

            pref = prefs[s]
            for g in range(CH // 16):
                v = plsc.load_gather(win, [lanesj + (g * 16 * NJ + j)])
                pref[pl.ds(g * 16, 16)] = v
            pltpu.async_copy(t64.at[pref], grows.at[s], gsems[s])

        def gather_wait(s):
            pltpu.make_async_copy(
                t64.at[prefs[s]], grows.at[s], gsems[s]).wait()

        def transpose(s):
            gref = grows.at[s]
            oref = oblk.at[s]

            @pl.loop(0, CH // 4)
            def _(kq):
                k0 = kq * 4
                ksplat0 = jnp.full((16,), k0, jnp.int32)
                for r in range(4):
                    ksplat = ksplat0 + r
                    for c in range(D // 16):
                        v = gref[k0 + r, pl.ds(c * 16, 16)]
                        plsc.store_scatter(oref, [lanes + c * 16, ksplat], v)

        def store_start(n, s):
            j, ib = decode(n)
            for tr in range(D // 8):
                pltpu.make_async_copy(
                    oblk.at[s, pl.ds(8 * tr, 8), pl.ds(0, CH)],
                    out.at[j, tr, ib], ssems[s]).start()

        def store_wait(s):
            for tr in range(D // 8):
                pltpu.make_async_copy(
                    oblk.at[s, pl.ds(8 * tr, 8), pl.ds(0, CH)],
                    out.at[0, tr, 0], ssems[s]).wait()

        prep(0, 0)

        @pl.loop(0, n_pairs)
        def _(h):
            n0 = 2 * h
            prep(n0 + 1, 1)
            gather_wait(0)

            @pl.when(h > 0)
            def _():
                store_wait(0)

            transpose(0)
            store_start(n0, 0)

            @pl.when(h < n_pairs - 1)
            def _():
                prep(n0 + 2, 0)

            gather_wait(1)

            @pl.when(h > 0)
            def _():
                store_wait(1)

            transpose(1)
            store_start(n0 + 1, 1)

        store_wait(0)
        store_wait(1)

    return k


@jax.jit
def kernel(data, table):
    NI, NJ = data.shape
    V, D = table.shape
    idx1d = data.reshape(NI * NJ)
    CH = 128
    out_t = _lookup_kernel(NI, NJ, D, CH)(idx1d, table)
    out_phys = out_t.transpose(0, 1, 3, 2, 4).reshape(NJ, D, NI)
    return out_phys.transpose(2, 0, 1)


# speedup vs baseline: 1.4203x; 1.0027x over previous
import functools

import jax
import jax.numpy as jnp
from jax import lax
from jax.experimental import pallas as pl
from jax.experimental.pallas import tpu as pltpu
from jax.experimental.pallas import tpu_sc as plsc


def _lookup_kernel(NI, NJ, D, CH):
    info = plsc.get_sparse_core_info()
    NC, NS = info.num_cores, info.num_subcores
    NW = NC * NS
    NB_I = NI // CH
    per_w = NB_I // NW
    n_sub = per_w * NJ
    n_pairs = n_sub // 2
    W = CH * NJ
    PITCH = CH + 1
    assert NI % CH == 0 and NB_I % NW == 0 and NJ % 2 == 0
    mesh = plsc.VectorSubcoreMesh(core_axis_name="c", subcore_axis_name="s")

    @functools.partial(
        pl.kernel,
        out_type=jax.ShapeDtypeStruct((NJ, D // 8, NB_I, 8, CH), jnp.float32),
        mesh=mesh,
        scratch_types=[
            pltpu.VMEM((W,), jnp.int32),
            pltpu.VMEM((CH,), jnp.int32),
            pltpu.VMEM((CH,), jnp.int32),
            pltpu.VMEM((CH,), jnp.int32),
            pltpu.VMEM((CH,), jnp.int32),
            pltpu.VMEM((4, CH, D), jnp.float32),
            pltpu.VMEM((2, D, PITCH), jnp.float32),
            pltpu.SemaphoreType.DMA,
            pltpu.SemaphoreType.DMA,
            pltpu.SemaphoreType.DMA,
            pltpu.SemaphoreType.DMA,
            pltpu.SemaphoreType.DMA,
            pltpu.SemaphoreType.DMA,
        ],
        compiler_params=pltpu.CompilerParams(
            use_tc_tiling_on_sc=False, needs_layout_passes=False),
    )
    def k(idx1d, t64, out, win, p0, p1, p2, p3, grows, oblk,
          g0, g1, g2, g3, s0, s1):
        wid = lax.axis_index("s") * NC + lax.axis_index("c")
        prefs = (p0, p1, p2, p3)
        gsems = (g0, g1, g2, g3)
        ssems = (s0, s1)
        lanes = lax.iota(jnp.int32, 16)
        lanesj = lanes * NJ

        def decode(n):
            m = n // NJ
            j = n - m * NJ
            return j, wid + m * NW

        def prep(n, s):
            j, ib = decode(n)

            @pl.when(j == 0)
            def _():
                pltpu.sync_copy(idx1d.at[pl.ds(ib * W, W)], win)

            pref = prefs[s]
            for g in range(CH // 16):
                v = plsc.load_gather(win, [lanesj + (g * 16 * NJ + j)])
                pref[pl.ds(g * 16, 16)] = v
            pltpu.async_copy(t64.at[pref], grows.at[s], gsems[s])

        def gather_wait(s):
            pltpu.make_async_copy(
                t64.at[prefs[s]], grows.at[s], gsems[s]).wait()

        def transpose(s, so):
            gref = grows.at[s]
            oref = oblk.at[so]

            @pl.loop(0, CH // 4)
            def _(kq):
                k0 = kq * 4
                ksplat0 = jnp.full((16,), k0, jnp.int32)
                for r in range(4):
                    ksplat = ksplat0 + r
                    for c in range(D // 16):
                        v = gref[k0 + r, pl.ds(c * 16, 16)]
                        plsc.store_scatter(oref, [lanes + c * 16, ksplat], v)

        def store_start(n, s):
            j, ib = decode(n)
            for tr in range(D // 8):
                pltpu.make_async_copy(
                    oblk.at[s, pl.ds(8 * tr, 8), pl.ds(0, CH)],
                    out.at[j, tr, ib], ssems[s]).start()

        def store_wait(s):
            for tr in range(D // 8):
                pltpu.make_async_copy(
                    oblk.at[s, pl.ds(8 * tr, 8), pl.ds(0, CH)],
                    out.at[0, tr, 0], ssems[s]).wait()

        prep(0, 0)
        prep(1, 1)
        prep(2, 2)

        @pl.loop(0, n_sub // 4)
        def _(q):
            for r in range(4):
                n = 4 * q + r
                so = r & 1
                gather_wait(r)

                @pl.when(n >= 2)
                def _():
                    store_wait(so)

                transpose(r, so)
                store_start(n, so)

                @pl.when(n + 3 < n_sub)
                def _():
                    prep(n + 3, (r + 3) % 4)

        store_wait(0)
        store_wait(1)

    return k


@jax.jit
def kernel(data, table):
    NI, NJ = data.shape
    V, D = table.shape
    idx1d = data.reshape(NI * NJ)
    CH = 128
    out_t = _lookup_kernel(NI, NJ, D, CH)(idx1d, table)
    out_phys = out_t.transpose(0, 1, 3, 2, 4).reshape(NJ, D, NI)
    return out_phys.transpose(2, 0, 1)
